# baseline (speedup 1.0000x reference)
import functools

import jax
import jax.numpy as jnp
from jax import lax
from jax.experimental import pallas as pl
from jax.experimental.pallas import tpu as pltpu
from jax.experimental.pallas import tpu_sc as plsc

N = 10000
E = 320000
D = 128
H = 256
C = 64

NC = 2
NS = 16
LN = 16
DS = 64
NP = N // 2

K = 128
NCHUNK = 160
TPT = NCHUNK * K
EPAD = TPT * NS
SUP = 8
SUPK = SUP * K
NSUP = NCHUNK // SUP
RPT = 312
REM = NP - NS * RPT

_mesh = plsc.VectorSubcoreMesh(core_axis_name="c", subcore_axis_name="s")

_GDN = lax.GatherDimensionNumbers(
    offset_dims=(), collapsed_slice_dims=(0,), start_index_map=(0,))


def _splat(vec16, e):
    idx = jnp.full((LN, 1), e, dtype=jnp.int32)
    return lax.gather(vec16, idx, _GDN, slice_sizes=(1,),
                      mode=lax.GatherScatterMode.PROMISE_IN_BOUNDS)


def _zero_rows0(rows):
    zero = jnp.zeros((LN,), jnp.float32)

    @pl.loop(0, K)
    def _(r):
        for f in range(D // LN):
            rows[0, r, pl.ds(f * LN, LN)] = zero


def _zero_acc(rows, acc_sh, s):
    base = s * RPT
    nfull = RPT // K
    rem = RPT - nfull * K
    for kk in range(nfull):
        pltpu.sync_copy(rows.at[0], acc_sh.at[pl.ds(base + kk * K, K)])
    if rem:
        pltpu.sync_copy(rows.at[0, pl.ds(0, rem)],
                        acc_sh.at[pl.ds(base + nfull * K, rem)])

    @pl.when(s == NS - 1)
    def _():
        pltpu.sync_copy(rows.at[0, pl.ds(0, REM)],
                        acc_sh.at[pl.ds(NS * RPT, REM)])


def _stage_src(src_stack, src_sh, s, p):
    pltpu.sync_copy(src_stack.at[pl.ds(p * NP + s * RPT, RPT)],
                    src_sh.at[pl.ds(s * RPT, RPT)])

    @pl.when(s == NS - 1)
    def _():
        pltpu.sync_copy(src_stack.at[pl.ds(p * NP + NS * RPT, REM)],
                        src_sh.at[pl.ds(NS * RPT, REM)])


def _writeout(acc_sh, out_hbm, s, out_base):
    pltpu.sync_copy(acc_sh.at[pl.ds(s * RPT, RPT)],
                    out_hbm.at[pl.ds(out_base + s * RPT, RPT)])

    @pl.when(s == NS - 1)
    def _():
        pltpu.sync_copy(acc_sh.at[pl.ds(NS * RPT, REM)],
                        out_hbm.at[pl.ds(out_base + NS * RPT, REM)])


def _edge_pass(src_sh, srcp_hbm, vlo_hbm, vhi_hbm, msrc_hbm, dstp_hbm,
               mbufs, rows, gsems, ssems, msems, acc_sh, tbase, rowbase):
    bsrcp, bvlo, bvhi, bmsrc, bdstp = mbufs

    def eload(sup_off, sl):
        off = tbase + sup_off
        pltpu.async_copy(srcp_hbm.at[pl.ds(off, SUPK)],
                         bsrcp.at[pl.ds(sl * SUPK, SUPK)], msems[sl])
        pltpu.async_copy(vlo_hbm.at[pl.ds(off, SUPK)],
                         bvlo.at[pl.ds(sl * SUPK, SUPK)], msems[sl])
        pltpu.async_copy(vhi_hbm.at[pl.ds(off, SUPK)],
                         bvhi.at[pl.ds(sl * SUPK, SUPK)], msems[sl])
        pltpu.async_copy(msrc_hbm.at[pl.ds(off, SUPK)],
                         bmsrc.at[pl.ds(sl * SUPK, SUPK)], msems[sl])
        pltpu.async_copy(dstp_hbm.at[pl.ds(rowbase + sup_off // K, SUP)],
                         bdstp.at[pl.ds(sl * SUP, SUP)], msems[sl])

    def eload_dyn(sup_idx, sl):
        off = tbase + sup_idx * SUPK
        pltpu.async_copy(srcp_hbm.at[pl.ds(off, SUPK)],
                         bsrcp.at[pl.ds(sl * SUPK, SUPK)], msems[sl])
        pltpu.async_copy(vlo_hbm.at[pl.ds(off, SUPK)],
                         bvlo.at[pl.ds(sl * SUPK, SUPK)], msems[sl])
        pltpu.async_copy(vhi_hbm.at[pl.ds(off, SUPK)],
                         bvhi.at[pl.ds(sl * SUPK, SUPK)], msems[sl])
        pltpu.async_copy(msrc_hbm.at[pl.ds(off, SUPK)],
                         bmsrc.at[pl.ds(sl * SUPK, SUPK)], msems[sl])
        pltpu.async_copy(dstp_hbm.at[pl.ds(rowbase + sup_idx * SUP, SUP)],
                         bdstp.at[pl.ds(sl * SUP, SUP)], msems[sl])

    def mwait(sl):
        pltpu.make_async_copy(srcp_hbm.at[pl.ds(tbase, SUPK)],
                              bsrcp.at[pl.ds(sl * SUPK, SUPK)],
                              msems[sl]).wait()
        pltpu.make_async_copy(vlo_hbm.at[pl.ds(tbase, SUPK)],
                              bvlo.at[pl.ds(sl * SUPK, SUPK)],
                              msems[sl]).wait()
        pltpu.make_async_copy(vhi_hbm.at[pl.ds(tbase, SUPK)],
                              bvhi.at[pl.ds(sl * SUPK, SUPK)],
                              msems[sl]).wait()
        pltpu.make_async_copy(msrc_hbm.at[pl.ds(tbase, SUPK)],
                              bmsrc.at[pl.ds(sl * SUPK, SUPK)],
                              msems[sl]).wait()
        pltpu.make_async_copy(dstp_hbm.at[pl.ds(rowbase, SUP)],
                              bdstp.at[pl.ds(sl * SUP, SUP)],
                              msems[sl]).wait()

    def start_gather(idx_start, m):
        pltpu.async_copy(src_sh.at[bsrcp.at[pl.ds(idx_start, K)]],
                         rows.at[m], gsems[m])

    def wait_gather(m):
        pltpu.make_async_copy(src_sh.at[bsrcp.at[pl.ds(0, K)]],
                              rows.at[m], gsems[m]).wait()

    def start_scatter(rowidx, m):
        pltpu.async_copy(rows.at[m], acc_sh.at[bdstp.at[rowidx]], ssems[m],
                         add=True)

    def wait_scatter(m):
        pltpu.make_async_copy(rows.at[m], acc_sh.at[bdstp.at[0]],
                              ssems[m]).wait()

    def scale(voff, m):
        @pl.loop(0, K // LN)
        def _(g):
            base = voff + g * LN
            vlo16 = bvlo[pl.ds(base, LN)]
            vhi16 = bvhi[pl.ds(base, LN)]
            ms16 = bmsrc[pl.ds(base, LN)]
            for e in range(LN):
                sl_v = _splat(vlo16, e)
                sh_v = _splat(vhi16, e)
                sm_v = _splat(ms16, e)
                r = g * LN + e
                for f in range(DS // LN):
                    lo = rows[m, r, pl.ds(f * LN, LN)]
                    hi = rows[m, r, pl.ds(DS + f * LN, LN)]
                    xsel = lo + sm_v * (hi - lo)
                    rows[m, r, pl.ds(f * LN, LN)] = xsel * sl_v
                    rows[m, r, pl.ds(DS + f * LN, LN)] = xsel * sh_v

    eload(0, 0)
    mwait(0)
    start_gather(0, 0)

    @pl.loop(0, NSUP // 2)
    def _(p2):
        for sl in (0, 1):
            @pl.loop(0, SUP // 2)
            def _(cp):
                for b in (0, 1):
                    i = (p2 * 2 + sl) * SUP + cp * 2 + b
                    wait_gather(b)
                    scale(sl * SUPK + (cp * 2 + b) * K, b)
                    start_scatter(sl * SUP + cp * 2 + b, b)

                    @pl.when(i + 1 < NCHUNK)
                    def _():
                        @pl.when(i >= 1)
                        def _():
                            wait_scatter(1 - b)
                        if b == 0:
                            @pl.when(jnp.logical_and(cp == 0,
                                                     i + SUP < NCHUNK))
                            def _():
                                eload_dyn(p2 * 2 + sl + 1, 1 - sl)
                            start_gather(sl * SUPK + (cp * 2 + 1) * K, 1)
                        else:
                            @pl.when(cp < SUP // 2 - 1)
                            def _():
                                start_gather(sl * SUPK + (cp * 2 + 2) * K, 0)

                            @pl.when(cp == SUP // 2 - 1)
                            def _():
                                mwait(1 - sl)
                                start_gather((1 - sl) * SUPK, 0)

    for m in range(2):
        wait_scatter(m)


def _make_spmm(P):
    scratch = [
        (pltpu.VMEM((2 * SUPK,), jnp.int32),
         pltpu.VMEM((2 * SUPK,), jnp.float32),
         pltpu.VMEM((2 * SUPK,), jnp.float32),
         pltpu.VMEM((2 * SUPK,), jnp.float32),
         pltpu.VMEM((2 * SUP, K), jnp.int32)),
        pltpu.VMEM((2, K, D), jnp.float32),
        pltpu.VMEM_SHARED((NP, D), jnp.float32),
        pltpu.VMEM_SHARED((NP, D), jnp.float32),
        [pltpu.SemaphoreType.DMA] * 2,
        [pltpu.SemaphoreType.DMA] * 2,
        [pltpu.SemaphoreType.DMA] * 2,
    ]

    @functools.partial(
        pl.kernel,
        out_type=jax.ShapeDtypeStruct((NC * P * NP, D), jnp.float32),
        mesh=_mesh,
        scratch_types=scratch,
    )
    def spmm(src_stack, srcp_hbm, vlo_hbm, vhi_hbm, msrc_hbm, dstp_hbm,
             out_hbm, mbufs, rows, src_sh, acc_sh, gsems, ssems, msems):
        c = lax.axis_index("c")
        s = lax.axis_index("s")
        tbase = c * EPAD + s * TPT
        rowbase = (c * NS + s) * NCHUNK

        @pl.loop(0, P)
        def _(p):
            _stage_src(src_stack, src_sh, s, p)
            _zero_rows0(rows)
            _zero_acc(rows, acc_sh, s)
            plsc.subcore_barrier()
            _edge_pass(src_sh, srcp_hbm, vlo_hbm, vhi_hbm, msrc_hbm,
                       dstp_hbm, mbufs, rows, gsems, ssems, msems,
                       acc_sh, tbase, rowbase)
            plsc.subcore_barrier()
            _writeout(acc_sh, out_hbm, s, (c * P + p) * NP)
            plsc.subcore_barrier()

    return spmm


_spmm_l1 = _make_spmm(2)
_spmm_l2 = _make_spmm(4)


_BM = 2000


def _mm1_body(x_ref, a0, a1, b0, b1, w_ref, h_ref):
    acc = jnp.dot(x_ref[...], w_ref[0:D, :],
                  preferred_element_type=jnp.float32)
    for i, r in enumerate((a0, a1, b0, b1)):
        acc += jnp.dot(r[...], w_ref[D + i * DS:D + (i + 1) * DS, :],
                       preferred_element_type=jnp.float32)
    h_ref[...] = jnp.maximum(acc, 0.0)


def _mm1(x, a0, a1, b0, b1, w1):
    return pl.pallas_call(
        _mm1_body,
        grid=(N // _BM,),
        in_specs=[pl.BlockSpec((_BM, D), lambda i: (i, 0))]
        + [pl.BlockSpec((_BM, DS), lambda i: (i, 0))] * 4
        + [pl.BlockSpec((3 * D, H), lambda i: (0, 0))],
        out_specs=pl.BlockSpec((_BM, H), lambda i: (i, 0)),
        out_shape=jax.ShapeDtypeStruct((N, H), jnp.float32),
    )(x, a0, a1, b0, b1, w1)


def _mm2_body(h_ref, c0, c1, c2, c3, d0, d1, d2, d3, w_ref, o_ref):
    acc = jnp.dot(h_ref[...], w_ref[0:H, :],
                  preferred_element_type=jnp.float32)
    for i, r in enumerate((c0, c1, c2, c3, d0, d1, d2, d3)):
        acc += jnp.dot(r[...], w_ref[H + i * DS:H + (i + 1) * DS, :],
                       preferred_element_type=jnp.float32)
    o_ref[...] = acc


def _mm2(h, cds, w2):
    return pl.pallas_call(
        _mm2_body,
        grid=(N // _BM,),
        in_specs=[pl.BlockSpec((_BM, H), lambda i: (i, 0))]
        + [pl.BlockSpec((_BM, DS), lambda i: (i, 0))] * 8
        + [pl.BlockSpec((3 * H, C), lambda i: (0, 0))],
        out_specs=pl.BlockSpec((_BM, C), lambda i: (i, 0)),
        out_shape=jax.ShapeDtypeStruct((N, C), jnp.float32),
    )(h, *cds, w2)


def _prep_edges(edge_index, values):
    pad = EPAD - E
    src = jnp.concatenate([edge_index[0], jnp.zeros((pad,), jnp.int32)])
    dst = jnp.concatenate([edge_index[1], jnp.zeros((pad,), jnp.int32)])
    val = jnp.concatenate([values, jnp.zeros((pad,), jnp.float32)])
    srcp = src >> 1
    msrc = (src & 1).astype(jnp.float32)
    hd = (dst & 1).astype(jnp.float32)
    vlo = val * (1.0 - hd)
    vhi = val * hd
    dstp = (dst >> 1).reshape(NS * NCHUNK, K)
    return srcp, msrc, vlo, vhi, dstp


def _pairs(a64):
    return a64.reshape(NP, D)


def _unpairs(blk):
    return blk.reshape(N, DS)


@jax.jit
def kernel(inputs, L_edge_index, L_values, L3_edge_index, L3_values, W1, W2):
    mL = _prep_edges(L_edge_index, L_values)
    mL3 = _prep_edges(L3_edge_index, L3_values)
    srcp, msrc, vlo, vhi, dstp = (
        jnp.concatenate([a, b]) for a, b in zip(mL, mL3))

    src1 = jnp.concatenate([_pairs(inputs[:, :DS]), _pairs(inputs[:, DS:])])
    ab = _spmm_l1(src1, srcp, vlo, vhi, msrc, dstp)
    h = _mm1(inputs, *[_unpairs(ab[k * NP:(k + 1) * NP]) for k in range(4)],
             W1)
    src2 = jnp.concatenate(
        [_pairs(h[:, k * DS:(k + 1) * DS]) for k in range(4)])
    cd = _spmm_l2(src2, srcp, vlo, vhi, msrc, dstp)
    out = _mm2(h, [_unpairs(cd[k * NP:(k + 1) * NP]) for k in range(8)], W2)
    return out


# speedup vs baseline: 1.2012x; 1.2012x over previous
import functools

import jax
import jax.numpy as jnp
from jax import lax
from jax.experimental import pallas as pl
from jax.experimental.pallas import tpu as pltpu
from jax.experimental.pallas import tpu_sc as plsc

N = 10000
E = 320000
D = 128
H = 256
C = 64

NC = 2
NS = 16
LN = 16
DS = 64
NP = N // 2

K = 128
NCHUNK = 160
TPT = NCHUNK * K
EPAD = TPT * NS
SUP = 8
SUPK = SUP * K
NSUP = NCHUNK // SUP
RPT = 312
REM = NP - NS * RPT

_mesh = plsc.VectorSubcoreMesh(core_axis_name="c", subcore_axis_name="s")

_GDN = lax.GatherDimensionNumbers(
    offset_dims=(), collapsed_slice_dims=(0,), start_index_map=(0,))


def _splat(vec16, e):
    idx = jnp.full((LN, 1), e, dtype=jnp.int32)
    return lax.gather(vec16, idx, _GDN, slice_sizes=(1,),
                      mode=lax.GatherScatterMode.PROMISE_IN_BOUNDS)


def _zero_rows0(rows):
    zero = jnp.zeros((LN,), jnp.float32)

    @pl.loop(0, K)
    def _(r):
        for f in range(D // LN):
            rows[0, r, pl.ds(f * LN, LN)] = zero


def _zero_acc(rows, acc_sh, s):
    base = s * RPT
    nfull = RPT // K
    rem = RPT - nfull * K
    for kk in range(nfull):
        pltpu.sync_copy(rows.at[0], acc_sh.at[pl.ds(base + kk * K, K)])
    if rem:
        pltpu.sync_copy(rows.at[0, pl.ds(0, rem)],
                        acc_sh.at[pl.ds(base + nfull * K, rem)])

    @pl.when(s == NS - 1)
    def _():
        pltpu.sync_copy(rows.at[0, pl.ds(0, REM)],
                        acc_sh.at[pl.ds(NS * RPT, REM)])


def _stage_src(src_stack, src_sh, s, p):
    pltpu.sync_copy(src_stack.at[pl.ds(p * NP + s * RPT, RPT)],
                    src_sh.at[pl.ds(s * RPT, RPT)])

    @pl.when(s == NS - 1)
    def _():
        pltpu.sync_copy(src_stack.at[pl.ds(p * NP + NS * RPT, REM)],
                        src_sh.at[pl.ds(NS * RPT, REM)])


def _writeout(acc_sh, out_hbm, s, out_base):
    pltpu.sync_copy(acc_sh.at[pl.ds(s * RPT, RPT)],
                    out_hbm.at[pl.ds(out_base + s * RPT, RPT)])

    @pl.when(s == NS - 1)
    def _():
        pltpu.sync_copy(acc_sh.at[pl.ds(NS * RPT, REM)],
                        out_hbm.at[pl.ds(out_base + NS * RPT, REM)])


def _edge_pass(src_sh, srcp_hbm, vlo_hbm, vhi_hbm, msrc_hbm, dstp_hbm,
               mbufs, rows, gsems, ssems, msems, acc_sh, tbase, rowbase):
    bsrcp, bvlo, bvhi, bmsrc, bdstp = mbufs

    def eload(sup_off, sl):
        off = tbase + sup_off
        pltpu.async_copy(srcp_hbm.at[pl.ds(off, SUPK)],
                         bsrcp.at[pl.ds(sl * SUPK, SUPK)], msems[sl])
        pltpu.async_copy(vlo_hbm.at[pl.ds(off, SUPK)],
                         bvlo.at[pl.ds(sl * SUPK, SUPK)], msems[sl])
        pltpu.async_copy(vhi_hbm.at[pl.ds(off, SUPK)],
                         bvhi.at[pl.ds(sl * SUPK, SUPK)], msems[sl])
        pltpu.async_copy(msrc_hbm.at[pl.ds(off, SUPK)],
                         bmsrc.at[pl.ds(sl * SUPK, SUPK)], msems[sl])
        pltpu.async_copy(dstp_hbm.at[pl.ds(rowbase + sup_off // K, SUP)],
                         bdstp.at[pl.ds(sl * SUP, SUP)], msems[sl])

    def eload_dyn(sup_idx, sl):
        off = tbase + sup_idx * SUPK
        pltpu.async_copy(srcp_hbm.at[pl.ds(off, SUPK)],
                         bsrcp.at[pl.ds(sl * SUPK, SUPK)], msems[sl])
        pltpu.async_copy(vlo_hbm.at[pl.ds(off, SUPK)],
                         bvlo.at[pl.ds(sl * SUPK, SUPK)], msems[sl])
        pltpu.async_copy(vhi_hbm.at[pl.ds(off, SUPK)],
                         bvhi.at[pl.ds(sl * SUPK, SUPK)], msems[sl])
        pltpu.async_copy(msrc_hbm.at[pl.ds(off, SUPK)],
                         bmsrc.at[pl.ds(sl * SUPK, SUPK)], msems[sl])
        pltpu.async_copy(dstp_hbm.at[pl.ds(rowbase + sup_idx * SUP, SUP)],
                         bdstp.at[pl.ds(sl * SUP, SUP)], msems[sl])

    def mwait(sl):
        pltpu.make_async_copy(srcp_hbm.at[pl.ds(tbase, SUPK)],
                              bsrcp.at[pl.ds(sl * SUPK, SUPK)],
                              msems[sl]).wait()
        pltpu.make_async_copy(vlo_hbm.at[pl.ds(tbase, SUPK)],
                              bvlo.at[pl.ds(sl * SUPK, SUPK)],
                              msems[sl]).wait()
        pltpu.make_async_copy(vhi_hbm.at[pl.ds(tbase, SUPK)],
                              bvhi.at[pl.ds(sl * SUPK, SUPK)],
                              msems[sl]).wait()
        pltpu.make_async_copy(msrc_hbm.at[pl.ds(tbase, SUPK)],
                              bmsrc.at[pl.ds(sl * SUPK, SUPK)],
                              msems[sl]).wait()
        pltpu.make_async_copy(dstp_hbm.at[pl.ds(rowbase, SUP)],
                              bdstp.at[pl.ds(sl * SUP, SUP)],
                              msems[sl]).wait()

    def start_gather(idx_start, m):
        pltpu.async_copy(src_sh.at[bsrcp.at[pl.ds(idx_start, K)]],
                         rows.at[m], gsems[m])

    def wait_gather(m):
        pltpu.make_async_copy(src_sh.at[bsrcp.at[pl.ds(0, K)]],
                              rows.at[m], gsems[m]).wait()

    def start_scatter(rowidx, m):
        pltpu.async_copy(rows.at[m], acc_sh.at[bdstp.at[rowidx]], ssems[m],
                         add=True)

    def wait_scatter(m):
        pltpu.make_async_copy(rows.at[m], acc_sh.at[bdstp.at[0]],
                              ssems[m]).wait()

    def scale(voff, m):
        @pl.loop(0, K // LN)
        def _(g):
            base = voff + g * LN
            vlo16 = bvlo[pl.ds(base, LN)]
            vhi16 = bvhi[pl.ds(base, LN)]
            ms16 = bmsrc[pl.ds(base, LN)]
            for e in range(LN):
                sl_v = _splat(vlo16, e)
                sh_v = _splat(vhi16, e)
                sm_v = _splat(ms16, e)
                r = g * LN + e
                for f in range(DS // LN):
                    lo = rows[m, r, pl.ds(f * LN, LN)]
                    hi = rows[m, r, pl.ds(DS + f * LN, LN)]
                    xsel = lo + sm_v * (hi - lo)
                    rows[m, r, pl.ds(f * LN, LN)] = xsel * sl_v
                    rows[m, r, pl.ds(DS + f * LN, LN)] = xsel * sh_v

    eload(0, 0)
    mwait(0)
    start_gather(0, 0)

    @pl.loop(0, NSUP // 2)
    def _(p2):
        for sl in (0, 1):
            @pl.loop(0, SUP // 2)
            def _(cp):
                for b in (0, 1):
                    i = (p2 * 2 + sl) * SUP + cp * 2 + b
                    wait_gather(b)

                    @pl.when(i + 1 < NCHUNK)
                    def _():
                        @pl.when(i >= 1)
                        def _():
                            wait_scatter(1 - b)
                        if b == 0:
                            @pl.when(jnp.logical_and(cp == 0,
                                                     i + SUP < NCHUNK))
                            def _():
                                eload_dyn(p2 * 2 + sl + 1, 1 - sl)
                            start_gather(sl * SUPK + (cp * 2 + 1) * K, 1)
                        else:
                            @pl.when(cp < SUP // 2 - 1)
                            def _():
                                start_gather(sl * SUPK + (cp * 2 + 2) * K, 0)

                            @pl.when(cp == SUP // 2 - 1)
                            def _():
                                mwait(1 - sl)
                                start_gather((1 - sl) * SUPK, 0)

                    scale(sl * SUPK + (cp * 2 + b) * K, b)
                    start_scatter(sl * SUP + cp * 2 + b, b)

    for m in range(2):
        wait_scatter(m)


def _make_spmm(P):
    scratch = [
        (pltpu.VMEM((2 * SUPK,), jnp.int32),
         pltpu.VMEM((2 * SUPK,), jnp.float32),
         pltpu.VMEM((2 * SUPK,), jnp.float32),
         pltpu.VMEM((2 * SUPK,), jnp.float32),
         pltpu.VMEM((2 * SUP, K), jnp.int32)),
        pltpu.VMEM((2, K, D), jnp.float32),
        pltpu.VMEM_SHARED((NP, D), jnp.float32),
        pltpu.VMEM_SHARED((NP, D), jnp.float32),
        [pltpu.SemaphoreType.DMA] * 2,
        [pltpu.SemaphoreType.DMA] * 2,
        [pltpu.SemaphoreType.DMA] * 2,
    ]

    @functools.partial(
        pl.kernel,
        out_type=jax.ShapeDtypeStruct((NC * P * NP, D), jnp.float32),
        mesh=_mesh,
        scratch_types=scratch,
    )
    def spmm(src_stack, srcp_hbm, vlo_hbm, vhi_hbm, msrc_hbm, dstp_hbm,
             out_hbm, mbufs, rows, src_sh, acc_sh, gsems, ssems, msems):
        c = lax.axis_index("c")
        s = lax.axis_index("s")
        tbase = c * EPAD + s * TPT
        rowbase = (c * NS + s) * NCHUNK

        @pl.loop(0, P)
        def _(p):
            _stage_src(src_stack, src_sh, s, p)
            _zero_rows0(rows)
            _zero_acc(rows, acc_sh, s)
            plsc.subcore_barrier()
            _edge_pass(src_sh, srcp_hbm, vlo_hbm, vhi_hbm, msrc_hbm,
                       dstp_hbm, mbufs, rows, gsems, ssems, msems,
                       acc_sh, tbase, rowbase)
            plsc.subcore_barrier()
            _writeout(acc_sh, out_hbm, s, (c * P + p) * NP)
            plsc.subcore_barrier()

    return spmm


_spmm_l1 = _make_spmm(2)
_spmm_l2 = _make_spmm(4)


_BM = 2000


def _mm1_body(x_ref, a0, a1, b0, b1, w_ref, h_ref):
    acc = jnp.dot(x_ref[...], w_ref[0:D, :],
                  preferred_element_type=jnp.float32)
    for i, r in enumerate((a0, a1, b0, b1)):
        acc += jnp.dot(r[...], w_ref[D + i * DS:D + (i + 1) * DS, :],
                       preferred_element_type=jnp.float32)
    h_ref[...] = jnp.maximum(acc, 0.0)


def _mm1(x, a0, a1, b0, b1, w1):
    return pl.pallas_call(
        _mm1_body,
        grid=(N // _BM,),
        in_specs=[pl.BlockSpec((_BM, D), lambda i: (i, 0))]
        + [pl.BlockSpec((_BM, DS), lambda i: (i, 0))] * 4
        + [pl.BlockSpec((3 * D, H), lambda i: (0, 0))],
        out_specs=pl.BlockSpec((_BM, H), lambda i: (i, 0)),
        out_shape=jax.ShapeDtypeStruct((N, H), jnp.float32),
    )(x, a0, a1, b0, b1, w1)


def _mm2_body(h_ref, c0, c1, c2, c3, d0, d1, d2, d3, w_ref, o_ref):
    acc = jnp.dot(h_ref[...], w_ref[0:H, :],
                  preferred_element_type=jnp.float32)
    for i, r in enumerate((c0, c1, c2, c3, d0, d1, d2, d3)):
        acc += jnp.dot(r[...], w_ref[H + i * DS:H + (i + 1) * DS, :],
                       preferred_element_type=jnp.float32)
    o_ref[...] = acc


def _mm2(h, cds, w2):
    return pl.pallas_call(
        _mm2_body,
        grid=(N // _BM,),
        in_specs=[pl.BlockSpec((_BM, H), lambda i: (i, 0))]
        + [pl.BlockSpec((_BM, DS), lambda i: (i, 0))] * 8
        + [pl.BlockSpec((3 * H, C), lambda i: (0, 0))],
        out_specs=pl.BlockSpec((_BM, C), lambda i: (i, 0)),
        out_shape=jax.ShapeDtypeStruct((N, C), jnp.float32),
    )(h, *cds, w2)


def _prep_edges(edge_index, values):
    pad = EPAD - E
    src = jnp.concatenate([edge_index[0], jnp.zeros((pad,), jnp.int32)])
    dst = jnp.concatenate([edge_index[1], jnp.zeros((pad,), jnp.int32)])
    val = jnp.concatenate([values, jnp.zeros((pad,), jnp.float32)])
    srcp = src >> 1
    msrc = (src & 1).astype(jnp.float32)
    hd = (dst & 1).astype(jnp.float32)
    vlo = val * (1.0 - hd)
    vhi = val * hd
    dstp = (dst >> 1).reshape(NS * NCHUNK, K)
    return srcp, msrc, vlo, vhi, dstp


def _pairs(a64):
    return a64.reshape(NP, D)


def _unpairs(blk):
    return blk.reshape(N, DS)


@jax.jit
def kernel(inputs, L_edge_index, L_values, L3_edge_index, L3_values, W1, W2):
    mL = _prep_edges(L_edge_index, L_values)
    mL3 = _prep_edges(L3_edge_index, L3_values)
    srcp, msrc, vlo, vhi, dstp = (
        jnp.concatenate([a, b]) for a, b in zip(mL, mL3))

    src1 = jnp.concatenate([_pairs(inputs[:, :DS]), _pairs(inputs[:, DS:])])
    ab = _spmm_l1(src1, srcp, vlo, vhi, msrc, dstp)
    h = _mm1(inputs, *[_unpairs(ab[k * NP:(k + 1) * NP]) for k in range(4)],
             W1)
    src2 = jnp.concatenate(
        [_pairs(h[:, k * DS:(k + 1) * DS]) for k in range(4)])
    cd = _spmm_l2(src2, srcp, vlo, vhi, msrc, dstp)
    out = _mm2(h, [_unpairs(cd[k * NP:(k + 1) * NP]) for k in range(8)], W2)
    return out
